# baseline (speedup 1.0000x reference)
import jax
import jax.numpy as jnp
from jax.experimental import pallas as pl
from jax.experimental.pallas import tpu as pltpu

_NCHUNK = 8


def _copy(x_ref, o_ref, sems):
    rows = x_ref.shape[0]
    blk = rows // _NCHUNK
    copies = []
    for i in range(_NCHUNK):
        c = pltpu.make_async_copy(
            x_ref.at[pl.ds(i * blk, blk)],
            o_ref.at[pl.ds(i * blk, blk)],
            sems.at[i],
        )
        c.start()
        copies.append(c)
    for c in copies:
        c.wait()


def kernel(X):
    b, n, f = X.shape
    rows = b * f
    flat = X.transpose(0, 2, 1).reshape(rows, n)
    out = pl.pallas_call(
        _copy,
        in_specs=[pl.BlockSpec(memory_space=pl.ANY)],
        out_specs=pl.BlockSpec(memory_space=pl.ANY),
        out_shape=jax.ShapeDtypeStruct((rows, n), X.dtype),
        scratch_shapes=[
            pltpu.SemaphoreType.DMA((_NCHUNK,)),
        ],
    )(flat)
    return out.reshape(b, f, n).transpose(0, 2, 1)


# speedup vs baseline: 27.2724x; 27.2724x over previous
import jax
import jax.numpy as jnp
from jax.experimental import pallas as pl
from jax.experimental.pallas import tpu as pltpu

_NCHUNK = 16


def _deep_copy(x_ref, o_ref, vmem, in_sems, out_sems):
    rows = x_ref.shape[0]
    blk = rows // _NCHUNK
    ins = []
    for i in range(_NCHUNK):
        c = pltpu.make_async_copy(
            x_ref.at[pl.ds(i * blk, blk)],
            vmem.at[pl.ds(i * blk, blk)],
            in_sems.at[i],
        )
        c.start()
        ins.append(c)
    outs = []
    for i in range(_NCHUNK):
        ins[i].wait()
        c = pltpu.make_async_copy(
            vmem.at[pl.ds(i * blk, blk)],
            o_ref.at[pl.ds(i * blk, blk)],
            out_sems.at[i],
        )
        c.start()
        outs.append(c)
    for c in outs:
        c.wait()


def kernel(X):
    b, n, f = X.shape
    rows = b * f
    flat = X.transpose(0, 2, 1).reshape(rows, n)
    out = pl.pallas_call(
        _deep_copy,
        in_specs=[pl.BlockSpec(memory_space=pl.ANY)],
        out_specs=pl.BlockSpec(memory_space=pl.ANY),
        out_shape=jax.ShapeDtypeStruct((rows, n), X.dtype),
        scratch_shapes=[
            pltpu.VMEM((rows, n), X.dtype),
            pltpu.SemaphoreType.DMA((_NCHUNK,)),
            pltpu.SemaphoreType.DMA((_NCHUNK,)),
        ],
    )(flat)
    return out.reshape(b, f, n).transpose(0, 2, 1)


# speedup vs baseline: 28.2863x; 1.0372x over previous
import jax
import jax.numpy as jnp
from jax.experimental import pallas as pl
from jax.experimental.pallas import tpu as pltpu

_NCHUNK = 4


def _deep_copy(x_ref, o_ref, vmem, in_sems, out_sems):
    rows = x_ref.shape[0]
    blk = rows // _NCHUNK
    ins = []
    for i in range(_NCHUNK):
        c = pltpu.make_async_copy(
            x_ref.at[pl.ds(i * blk, blk)],
            vmem.at[pl.ds(i * blk, blk)],
            in_sems.at[i],
        )
        c.start()
        ins.append(c)
    outs = []
    for i in range(_NCHUNK):
        ins[i].wait()
        c = pltpu.make_async_copy(
            vmem.at[pl.ds(i * blk, blk)],
            o_ref.at[pl.ds(i * blk, blk)],
            out_sems.at[i],
        )
        c.start()
        outs.append(c)
    for c in outs:
        c.wait()


def kernel(X):
    b, n, f = X.shape
    rows = b * f
    flat = X.transpose(0, 2, 1).reshape(rows, n)
    out = pl.pallas_call(
        _deep_copy,
        in_specs=[pl.BlockSpec(memory_space=pl.ANY)],
        out_specs=pl.BlockSpec(memory_space=pl.ANY),
        out_shape=jax.ShapeDtypeStruct((rows, n), X.dtype),
        scratch_shapes=[
            pltpu.VMEM((rows, n), X.dtype),
            pltpu.SemaphoreType.DMA((_NCHUNK,)),
            pltpu.SemaphoreType.DMA((_NCHUNK,)),
        ],
    )(flat)
    return out.reshape(b, f, n).transpose(0, 2, 1)


# speedup vs baseline: 28.8914x; 1.0214x over previous
import jax
import jax.numpy as jnp
from jax.experimental import pallas as pl
from jax.experimental.pallas import tpu as pltpu

_NCHUNK = 8


def _deep_copy(x_ref, o_ref, vmem, in_sems, out_sems):
    rows = x_ref.shape[0]
    blk = rows // _NCHUNK
    ins = []
    for i in range(_NCHUNK):
        c = pltpu.make_async_copy(
            x_ref.at[pl.ds(i * blk, blk)],
            vmem.at[pl.ds(i * blk, blk)],
            in_sems.at[i],
        )
        c.start()
        ins.append(c)
    outs = []
    for i in range(_NCHUNK):
        ins[i].wait()
        c = pltpu.make_async_copy(
            vmem.at[pl.ds(i * blk, blk)],
            o_ref.at[pl.ds(i * blk, blk)],
            out_sems.at[i],
        )
        c.start()
        outs.append(c)
    for c in outs:
        c.wait()


def kernel(X):
    b, n, f = X.shape
    rows = b * f
    flat = X.transpose(0, 2, 1).reshape(rows, n)
    out = pl.pallas_call(
        _deep_copy,
        in_specs=[pl.BlockSpec(memory_space=pl.ANY)],
        out_specs=pl.BlockSpec(memory_space=pl.ANY),
        out_shape=jax.ShapeDtypeStruct((rows, n), X.dtype),
        scratch_shapes=[
            pltpu.VMEM((rows, n), X.dtype),
            pltpu.SemaphoreType.DMA((_NCHUNK,)),
            pltpu.SemaphoreType.DMA((_NCHUNK,)),
        ],
    )(flat)
    return out.reshape(b, f, n).transpose(0, 2, 1)
